# initial kernel scaffold (unmeasured)
import jax
import jax.numpy as jnp
from jax import lax
from jax.experimental import pallas as pl
from jax.experimental.pallas import tpu as pltpu

N_DEV = 4
B, H, D, BS = 8, 8, 64, 16
NB = 64
PAGES = 64
NKEYS = PAGES * BS
ROWS = B * H
CBLK = 2 * ROWS


def kernel(Q, K, V, bt, lens):
    Qr = Q.reshape(B, H, D)
    Kr = K.reshape(NKEYS, H, D)
    Vr = V.reshape(NKEYS, H, D)
    lr = lens.reshape(B, 1)

    def body(q_ref, k_ref, v_ref, bt_ref, lens_ref, out_ref,
             comm_ref, recv_ref, send_sems, recv_sems):
        my = lax.axis_index("i")
        left = (my + N_DEV - 1) % N_DEV
        right = (my + 1) % N_DEV

        barrier = pltpu.get_barrier_semaphore()
        pl.semaphore_signal(barrier, inc=1, device_id=(left,),
                            device_id_type=pl.DeviceIdType.MESH)
        pl.semaphore_signal(barrier, inc=1, device_id=(right,),
                            device_id_type=pl.DeviceIdType.MESH)
        pl.semaphore_wait(barrier, 2)

        rel = bt_ref[:, :] - my * PAGES
        col = lax.broadcasted_iota(jnp.int32, (B, NB), 1)
        valid = (col < lens_ref[:, :]) & (rel >= 0) & (rel < PAGES)
        page_iota = lax.broadcasted_iota(jnp.int32, (B, NB, PAGES), 2)
        onehot = (rel[:, :, None] == page_iota) & valid[:, :, None]
        counts = jnp.sum(onehot.astype(jnp.float32), axis=1)
        krow = lax.broadcasted_iota(jnp.int32, (PAGES, NKEYS), 0)
        kcol = lax.broadcasted_iota(jnp.int32, (PAGES, NKEYS), 1)
        expand = (kcol // BS == krow).astype(jnp.float32)
        w = lax.dot_general(counts, expand, (((1,), (0,)), ((), ())),
                            preferred_element_type=jnp.float32)

        scale = D ** -0.5
        for h in range(H):
            qh = (q_ref[:, h, :] * scale).astype(jnp.bfloat16)
            kh = k_ref[:, h, :].astype(jnp.bfloat16)
            s = lax.dot_general(qh, kh, (((1,), (1,)), ((), ())),
                                preferred_element_type=jnp.float32)
            p = jnp.exp(s) * w
            lh = jnp.sum(p, axis=1, keepdims=True)
            vh = v_ref[:, h, :].astype(jnp.bfloat16)
            acc = lax.dot_general(p.astype(jnp.bfloat16), vh,
                                  (((1,), (0,)), ((), ())),
                                  preferred_element_type=jnp.float32)
            comm_ref[h * B:(h + 1) * B, :] = acc
            comm_ref[ROWS + h * B:ROWS + (h + 1) * B, :] = (
                jnp.broadcast_to(lh, (B, D)))

        for hop in range(N_DEV - 1):
            src = comm_ref if hop == 0 else recv_ref.at[hop - 1]
            rdma = pltpu.make_async_remote_copy(
                src_ref=src,
                dst_ref=recv_ref.at[hop],
                send_sem=send_sems.at[hop],
                recv_sem=recv_sems.at[hop],
                device_id=(right,),
                device_id_type=pl.DeviceIdType.MESH,
            )
            rdma.start()
            rdma.wait()

        total = comm_ref[:, :] + recv_ref[0] + recv_ref[1] + recv_ref[2]
        for h in range(H):
            acc = total[h * B:(h + 1) * B, :]
            lsum = total[ROWS + h * B:ROWS + (h + 1) * B, :]
            out_ref[:, h, :] = acc / lsum

    out = pl.pallas_call(
        body,
        out_shape=jax.ShapeDtypeStruct((B, H, D), jnp.float32),
        in_specs=[pl.BlockSpec(memory_space=pltpu.VMEM)] * 5,
        out_specs=pl.BlockSpec(memory_space=pltpu.VMEM),
        scratch_shapes=[
            pltpu.VMEM((CBLK, D), jnp.float32),
            pltpu.VMEM((N_DEV - 1, CBLK, D), jnp.float32),
            pltpu.SemaphoreType.DMA((N_DEV - 1,)),
            pltpu.SemaphoreType.DMA((N_DEV - 1,)),
        ],
        compiler_params=pltpu.CompilerParams(collective_id=0),
    )(Qr, Kr, Vr, bt, lr)
    return out.reshape(B, 1, H, D)


# baseline (device time: 26326 ns/iter reference)
import jax
import jax.numpy as jnp
from jax import lax
from jax.experimental import pallas as pl
from jax.experimental.pallas import tpu as pltpu

N_DEV = 4
B, H, D, BS = 8, 8, 64, 16
NB = 64
PAGES = 64
NKEYS = PAGES * BS
ROWS = B * H
CBLK = 2 * ROWS


def kernel(Q, K, V, bt, lens):
    Qr = Q.reshape(B, H, D)
    Kr = K.reshape(NKEYS, H, D)
    Vr = V.reshape(NKEYS, H, D)
    lr = lens.reshape(B, 1)

    def body(q_ref, k_ref, v_ref, bt_ref, lens_ref, out_ref,
             comm_ref, recv_ref, send_sems, recv_sems):
        my = lax.axis_index("i")
        left = (my + N_DEV - 1) % N_DEV
        right = (my + 1) % N_DEV

        barrier = pltpu.get_barrier_semaphore()
        pl.semaphore_signal(barrier, inc=1, device_id=(left,),
                            device_id_type=pl.DeviceIdType.MESH)
        pl.semaphore_signal(barrier, inc=1, device_id=(right,),
                            device_id_type=pl.DeviceIdType.MESH)
        pl.semaphore_wait(barrier, 2)

        rel = bt_ref[:, :] - my * PAGES
        p_iota = lax.broadcasted_iota(jnp.int32, (B, PAGES), 1)
        counts = jnp.zeros((B, PAGES), jnp.float32)
        for j in range(NB):
            sel = (rel[:, j:j + 1] == p_iota) & (j < lens_ref[:, :])
            counts = counts + sel.astype(jnp.float32)
        krow = lax.broadcasted_iota(jnp.int32, (PAGES, NKEYS), 0)
        kcol = lax.broadcasted_iota(jnp.int32, (PAGES, NKEYS), 1)
        expand = (kcol // BS == krow).astype(jnp.float32)
        w = lax.dot_general(counts, expand, (((1,), (0,)), ((), ())),
                            preferred_element_type=jnp.float32)

        scale = D ** -0.5
        for h in range(H):
            qh = (q_ref[:, h, :] * scale).astype(jnp.bfloat16)
            kh = k_ref[:, h, :].astype(jnp.bfloat16)
            s = lax.dot_general(qh, kh, (((1,), (1,)), ((), ())),
                                preferred_element_type=jnp.float32)
            p = jnp.exp(s) * w
            lh = jnp.sum(p, axis=1, keepdims=True)
            vh = v_ref[:, h, :].astype(jnp.bfloat16)
            acc = lax.dot_general(p.astype(jnp.bfloat16), vh,
                                  (((1,), (0,)), ((), ())),
                                  preferred_element_type=jnp.float32)
            comm_ref[h * B:(h + 1) * B, :] = acc
            comm_ref[ROWS + h * B:ROWS + (h + 1) * B, :] = (
                jnp.broadcast_to(lh, (B, D)))

        for hop in range(N_DEV - 1):
            src = comm_ref if hop == 0 else recv_ref.at[hop - 1]
            rdma = pltpu.make_async_remote_copy(
                src_ref=src,
                dst_ref=recv_ref.at[hop],
                send_sem=send_sems.at[hop],
                recv_sem=recv_sems.at[hop],
                device_id=(right,),
                device_id_type=pl.DeviceIdType.MESH,
            )
            rdma.start()
            rdma.wait()

        total = comm_ref[:, :] + recv_ref[0] + recv_ref[1] + recv_ref[2]
        for h in range(H):
            acc = total[h * B:(h + 1) * B, :]
            lsum = total[ROWS + h * B:ROWS + (h + 1) * B, :]
            out_ref[:, h, :] = acc / lsum

    out = pl.pallas_call(
        body,
        out_shape=jax.ShapeDtypeStruct((B, H, D), jnp.float32),
        in_specs=[pl.BlockSpec(memory_space=pltpu.VMEM)] * 5,
        out_specs=pl.BlockSpec(memory_space=pltpu.VMEM),
        scratch_shapes=[
            pltpu.VMEM((CBLK, D), jnp.float32),
            pltpu.VMEM((N_DEV - 1, CBLK, D), jnp.float32),
            pltpu.SemaphoreType.DMA((N_DEV - 1,)),
            pltpu.SemaphoreType.DMA((N_DEV - 1,)),
        ],
        compiler_params=pltpu.CompilerParams(collective_id=0),
    )(Qr, Kr, Vr, bt, lr)
    return out.reshape(B, 1, H, D)


# device time: 13348 ns/iter; 1.9723x vs baseline; 1.9723x over previous
import jax
import jax.numpy as jnp
from jax import lax
from jax.experimental import pallas as pl
from jax.experimental.pallas import tpu as pltpu

N_DEV = 4
B, H, D, BS = 8, 8, 64, 16
NB = 64
PAGES = 64
NKEYS = PAGES * BS
ROWS = B * H
CBLK = 2 * ROWS


def kernel(Q, K, V, bt, lens):
    Q2 = Q.reshape(B, H * D)
    K2 = K.reshape(NKEYS, H * D)
    V2 = V.reshape(NKEYS, H * D)
    lr = lens.reshape(B, 1)

    def body(q_ref, k_ref, v_ref, bt_ref, lens_ref, out_ref,
             comm_ref, recv_ref, send_sems, recv_sems):
        my = lax.axis_index("i")

        barrier = pltpu.get_barrier_semaphore()
        for off in range(1, N_DEV):
            pl.semaphore_signal(barrier, inc=1,
                                device_id=((my + off) % N_DEV,),
                                device_id_type=pl.DeviceIdType.MESH)

        rel = bt_ref[:, :] - my * PAGES
        p_iota = lax.broadcasted_iota(jnp.int32, (B, PAGES), 1)
        counts = jnp.zeros((B, PAGES), jnp.float32)
        for j in range(NB):
            sel = (rel[:, j:j + 1] == p_iota) & (j < lens_ref[:, :])
            counts = counts + sel.astype(jnp.float32)
        krow = lax.broadcasted_iota(jnp.int32, (PAGES, NKEYS), 0)
        kcol = lax.broadcasted_iota(jnp.int32, (PAGES, NKEYS), 1)
        expand = (kcol // BS == krow).astype(jnp.float32)
        w8 = lax.dot_general(counts, expand, (((1,), (0,)), ((), ())),
                             preferred_element_type=jnp.float32)
        wf = jnp.concatenate([w8] * H, axis=0)

        scale = D ** -0.5
        qcat = jnp.concatenate(
            [q_ref[:, h * D:(h + 1) * D] for h in range(H)], axis=0
        ) * scale
        acat = jnp.concatenate([qcat] * H, axis=1)
        arow = lax.broadcasted_iota(jnp.int32, (ROWS, H * D), 0)
        acol = lax.broadcasted_iota(jnp.int32, (ROWS, H * D), 1)
        a = jnp.where(acol // D == arow // B, acat, 0.0).astype(jnp.bfloat16)

        s = lax.dot_general(a, k_ref[:, :].astype(jnp.bfloat16),
                            (((1,), (1,)), ((), ())),
                            preferred_element_type=jnp.float32)
        p = jnp.exp(s) * wf
        lh = jnp.sum(p, axis=1, keepdims=True)
        accbig = lax.dot_general(p.astype(jnp.bfloat16),
                                 v_ref[:, :].astype(jnp.bfloat16),
                                 (((1,), (0,)), ((), ())),
                                 preferred_element_type=jnp.float32)
        for h in range(H):
            comm_ref[h * B:(h + 1) * B, :] = (
                accbig[h * B:(h + 1) * B, h * D:(h + 1) * D]
                .astype(jnp.bfloat16))
        comm_ref[ROWS:CBLK, :] = (
            jnp.broadcast_to(lh, (ROWS, D)).astype(jnp.bfloat16))

        pl.semaphore_wait(barrier, N_DEV - 1)
        rdmas = {}
        for off in (2, 1, 3):
            rdma = pltpu.make_async_remote_copy(
                src_ref=comm_ref,
                dst_ref=recv_ref.at[off - 1],
                send_sem=send_sems.at[off - 1],
                recv_sem=recv_sems.at[off - 1],
                device_id=((my + off) % N_DEV,),
                device_id_type=pl.DeviceIdType.MESH,
            )
            rdma.start()
            rdmas[off] = rdma

        total = comm_ref[:, :].astype(jnp.float32)
        for off in (1, 3, 2):
            rdmas[off].wait_recv()
            total = total + recv_ref[off - 1].astype(jnp.float32)
        for off in (1, 3, 2):
            rdmas[off].wait_send()
        for h in range(H):
            acc = total[h * B:(h + 1) * B, :]
            lsum = total[ROWS + h * B:ROWS + (h + 1) * B, :]
            out_ref[:, h, :] = acc / lsum

    out = pl.pallas_call(
        body,
        out_shape=jax.ShapeDtypeStruct((B, H, D), jnp.float32),
        in_specs=[pl.BlockSpec(memory_space=pltpu.VMEM)] * 5,
        out_specs=pl.BlockSpec(memory_space=pltpu.VMEM),
        scratch_shapes=[
            pltpu.VMEM((CBLK, D), jnp.bfloat16),
            pltpu.VMEM((N_DEV - 1, CBLK, D), jnp.bfloat16),
            pltpu.SemaphoreType.DMA((N_DEV - 1,)),
            pltpu.SemaphoreType.DMA((N_DEV - 1,)),
        ],
        compiler_params=pltpu.CompilerParams(collective_id=0),
    )(Q2, K2, V2, bt, lr)
    return out.reshape(B, 1, H, D)
